# baseline (speedup 1.0000x reference)
import functools

import jax
import jax.numpy as jnp
from jax import lax
from jax.experimental import pallas as pl

ALPHA = 0.6
BETA = 0.2
GAMMA = 0.2
MASKING_RATIO = 0.5

LANES = 128


def _score_body(x_ref, g_ref, p_ref, s_ref):
    x = x_ref[0]
    e = jnp.sqrt(jnp.sum(x * x, axis=0))
    g = jax.nn.sigmoid(g_ref[0, 0])
    p = jax.nn.sigmoid(p_ref[0, 0])
    s_ref[0, 0] = ALPHA * e + BETA * g + GAMMA * p


def _select_body(s_ref, m_ref, *, k):
    s = s_ref[...]
    B, NR, L = s.shape
    n = NR * L
    bits = lax.bitcast_convert_type(s, jnp.int32)

    def count_ge(t):
        return jnp.sum((bits >= t).astype(jnp.int32), axis=(1, 2), keepdims=True)

    lo = jnp.zeros((B, 1, 1), jnp.int32)
    hi = jnp.full((B, 1, 1), 0x7F800000, jnp.int32)

    def bs_body(_, lohi):
        lo, hi = lohi
        mid = lo + (hi - lo + 1) // 2
        pred = count_ge(mid) >= k
        return jnp.where(pred, mid, lo), jnp.where(pred, hi, mid - 1)

    lo, hi = lax.fori_loop(0, 31, bs_body, (lo, hi))
    t = lo

    gt = bits > t
    cnt_gt = jnp.sum(gt.astype(jnp.int32), axis=(1, 2), keepdims=True)
    need = k - cnt_gt
    tie = bits == t
    idx = (
        lax.broadcasted_iota(jnp.int32, (B, NR, L), 1) * L
        + lax.broadcasted_iota(jnp.int32, (B, NR, L), 2)
    )

    lo_j = jnp.zeros((B, 1, 1), jnp.int32)
    hi_j = jnp.full((B, 1, 1), n, jnp.int32)

    def bs2_body(_, lohi):
        lo, hi = lohi
        mid = (lo + hi) // 2
        g = jnp.sum((tie & (idx < mid)).astype(jnp.int32), axis=(1, 2), keepdims=True)
        pred = g >= need
        return jnp.where(pred, lo, mid + 1), jnp.where(pred, mid, hi)

    lo_j, hi_j = lax.fori_loop(0, 18, bs2_body, (lo_j, hi_j))
    j = lo_j

    m_ref[...] = (gt | (tie & (idx < j))).astype(jnp.float32)


def _apply_body(x_ref, m_ref, o_ref):
    o_ref[...] = x_ref[...] * m_ref[...]


def kernel(x, gaze_importance, pose_importance):
    B, C, H, W = x.shape
    N = H * W
    k = int(MASKING_RATIO * N)

    NB = next(nb for nb in (6, 4, 2, 1) if H % nb == 0)
    BH = H // NB
    NBA = next(nb for nb in (4, 2, 1) if H % nb == 0)
    BHA = H // NBA

    gi4 = gaze_importance.reshape(B, 1, H, W)
    pi4 = pose_importance.reshape(B, 1, H, W)
    scores = pl.pallas_call(
        _score_body,
        grid=(B, NB),
        in_specs=[
            pl.BlockSpec((1, C, BH, W), lambda b, i: (b, 0, i, 0)),
            pl.BlockSpec((1, 1, BH, W), lambda b, i: (b, 0, i, 0)),
            pl.BlockSpec((1, 1, BH, W), lambda b, i: (b, 0, i, 0)),
        ],
        out_specs=pl.BlockSpec((1, 1, BH, W), lambda b, i: (b, 0, i, 0)),
        out_shape=jax.ShapeDtypeStruct((B, 1, H, W), jnp.float32),
    )(x, gi4, pi4)

    NR = N // LANES
    mask = pl.pallas_call(
        functools.partial(_select_body, k=k),
        in_specs=[pl.BlockSpec((B, NR, LANES), lambda: (0, 0, 0))],
        out_specs=pl.BlockSpec((B, NR, LANES), lambda: (0, 0, 0)),
        out_shape=jax.ShapeDtypeStruct((B, NR, LANES), jnp.float32),
    )(scores.reshape(B, NR, LANES))
    mask = mask.reshape(B, 1, H, W)

    out = pl.pallas_call(
        _apply_body,
        grid=(B, NBA),
        in_specs=[
            pl.BlockSpec((1, C, BHA, W), lambda b, i: (b, 0, i, 0)),
            pl.BlockSpec((1, 1, BHA, W), lambda b, i: (b, 0, i, 0)),
        ],
        out_specs=pl.BlockSpec((1, C, BHA, W), lambda b, i: (b, 0, i, 0)),
        out_shape=jax.ShapeDtypeStruct((B, C, H, W), jnp.float32),
    )(x, mask)

    return out


# speedup vs baseline: 1.0368x; 1.0368x over previous
import functools

import jax
import jax.numpy as jnp
from jax import lax
from jax.experimental import pallas as pl

ALPHA = 0.6
BETA = 0.2
GAMMA = 0.2
MASKING_RATIO = 0.5

LANES = 128


def _score_body(x_ref, g_ref, p_ref, s_ref):
    x = x_ref[0]
    e = jnp.sqrt(jnp.sum(x * x, axis=0))
    g = jax.nn.sigmoid(g_ref[0, 0])
    p = jax.nn.sigmoid(p_ref[0, 0])
    s_ref[0, 0] = ALPHA * e + BETA * g + GAMMA * p


def _select_body(s_ref, m_ref, *, k):
    s = s_ref[...]
    B, _, H, W = s.shape
    n = H * W
    bits = lax.bitcast_convert_type(s, jnp.int32)

    def count_ge(t):
        return jnp.sum((bits >= t).astype(jnp.int32), axis=(1, 2, 3), keepdims=True)

    lo = jnp.zeros((B, 1, 1, 1), jnp.int32)
    hi = jnp.full((B, 1, 1, 1), 0x7F800000, jnp.int32)

    def bs_body(_, lohi):
        lo, hi = lohi
        mid = lo + (hi - lo + 1) // 2
        pred = count_ge(mid) >= k
        return jnp.where(pred, mid, lo), jnp.where(pred, hi, mid - 1)

    lo, hi = lax.fori_loop(0, 31, bs_body, (lo, hi))
    t = lo

    gt = bits > t
    cnt_gt = jnp.sum(gt.astype(jnp.int32), axis=(1, 2, 3), keepdims=True)
    need = k - cnt_gt
    tie = bits == t
    idx = (
        lax.broadcasted_iota(jnp.int32, (B, 1, H, W), 2) * W
        + lax.broadcasted_iota(jnp.int32, (B, 1, H, W), 3)
    )

    lo_j = jnp.zeros((B, 1, 1, 1), jnp.int32)
    hi_j = jnp.full((B, 1, 1, 1), n, jnp.int32)

    def bs2_body(_, lohi):
        lo, hi = lohi
        mid = (lo + hi) // 2
        g = jnp.sum(
            (tie & (idx < mid)).astype(jnp.int32), axis=(1, 2, 3), keepdims=True
        )
        pred = g >= need
        return jnp.where(pred, lo, mid + 1), jnp.where(pred, mid, hi)

    lo_j, hi_j = lax.fori_loop(0, 18, bs2_body, (lo_j, hi_j))
    j = lo_j

    m_ref[...] = (gt | (tie & (idx < j))).astype(jnp.float32)


def _apply_body(x_ref, m_ref, o_ref):
    o_ref[...] = x_ref[...] * m_ref[...]


def kernel(x, gaze_importance, pose_importance):
    B, C, H, W = x.shape
    N = H * W
    k = int(MASKING_RATIO * N)

    NB = next(nb for nb in (6, 4, 2, 1) if H % nb == 0)
    BH = H // NB
    NBA = next(nb for nb in (4, 2, 1) if H % nb == 0)
    BHA = H // NBA

    gi4 = gaze_importance.reshape(B, 1, H, W)
    pi4 = pose_importance.reshape(B, 1, H, W)
    scores = pl.pallas_call(
        _score_body,
        grid=(B, NB),
        in_specs=[
            pl.BlockSpec((1, C, BH, W), lambda b, i: (b, 0, i, 0)),
            pl.BlockSpec((1, 1, BH, W), lambda b, i: (b, 0, i, 0)),
            pl.BlockSpec((1, 1, BH, W), lambda b, i: (b, 0, i, 0)),
        ],
        out_specs=pl.BlockSpec((1, 1, BH, W), lambda b, i: (b, 0, i, 0)),
        out_shape=jax.ShapeDtypeStruct((B, 1, H, W), jnp.float32),
    )(x, gi4, pi4)

    mask = pl.pallas_call(
        functools.partial(_select_body, k=k),
        in_specs=[pl.BlockSpec((B, 1, H, W), lambda: (0, 0, 0, 0))],
        out_specs=pl.BlockSpec((B, 1, H, W), lambda: (0, 0, 0, 0)),
        out_shape=jax.ShapeDtypeStruct((B, 1, H, W), jnp.float32),
    )(scores)

    out = pl.pallas_call(
        _apply_body,
        grid=(B, NBA),
        in_specs=[
            pl.BlockSpec((1, C, BHA, W), lambda b, i: (b, 0, i, 0)),
            pl.BlockSpec((1, 1, BHA, W), lambda b, i: (b, 0, i, 0)),
        ],
        out_specs=pl.BlockSpec((1, C, BHA, W), lambda b, i: (b, 0, i, 0)),
        out_shape=jax.ShapeDtypeStruct((B, C, H, W), jnp.float32),
    )(x, mask)

    return out
